# baseline (speedup 1.0000x reference)
import functools

import jax
import jax.numpy as jnp
from jax import lax
from jax.experimental import pallas as pl
from jax.experimental.pallas import tpu as pltpu
from jax.experimental.pallas import tpu_sc as plsc

DEPTH = 64
NUM_TOK = 8192
NUM_WORKERS = 32
TOK_PER_W = NUM_TOK // NUM_WORKERS
SEG_PER_ROW = 2048 // TOK_PER_W
CHUNK = 128
NCHUNK = TOK_PER_W // CHUNK


def _emb_body(idx_hbm, tok_hbm, pos_hbm, out_hbm, idx_v, rows_v, pos_v,
              gsem, psem):
    wid = lax.axis_index("s") * 2 + lax.axis_index("c")
    tok_base = wid * TOK_PER_W
    pos_base = (wid % SEG_PER_ROW) * TOK_PER_W

    pltpu.sync_copy(idx_hbm.at[pl.ds(wid * NCHUNK, NCHUNK)], idx_v)

    pcopy = pltpu.async_copy(pos_hbm.at[pl.ds(pos_base, TOK_PER_W)], pos_v,
                             psem)
    gcopies = [
        pltpu.async_copy(tok_hbm.at[idx_v.at[j]],
                         rows_v.at[pl.ds(j * CHUNK, CHUNK)], gsem)
        for j in range(NCHUNK)
    ]
    for c in gcopies:
        c.wait()
    pcopy.wait()

    def add_row(i, carry):
        for k in range(DEPTH // 16):
            sl = pl.ds(k * 16, 16)
            rows_v[i, sl] = rows_v[i, sl] + pos_v[i, sl]
        return carry

    lax.fori_loop(0, TOK_PER_W, add_row, 0)

    pltpu.sync_copy(rows_v, out_hbm.at[pl.ds(tok_base, TOK_PER_W)])


_emb_call = functools.partial(
    pl.kernel,
    mesh=plsc.VectorSubcoreMesh(core_axis_name="c", subcore_axis_name="s"),
    out_type=jax.ShapeDtypeStruct((NUM_TOK, DEPTH), jnp.float32),
    scratch_types=[
        pltpu.VMEM((NCHUNK, CHUNK), jnp.int32),
        pltpu.VMEM((TOK_PER_W, DEPTH), jnp.float32),
        pltpu.VMEM((TOK_PER_W, DEPTH), jnp.float32),
        pltpu.SemaphoreType.DMA,
        pltpu.SemaphoreType.DMA,
    ],
    compiler_params=pltpu.CompilerParams(use_tc_tiling_on_sc=False),
)(_emb_body)


def kernel(x, tok_table, pos_table):
    b, s = x.shape
    xf = x.reshape(NUM_TOK // CHUNK, CHUNK).astype(jnp.int32)
    out = _emb_call(xf, tok_table, pos_table)
    return out.reshape(b, s, DEPTH)
